# Initial kernel scaffold; baseline (speedup 1.0000x reference)
import jax
import jax.numpy as jnp
from jax.experimental import pallas as pl


def kernel(input, pool_idx):
    raise NotImplementedError("write your pallas kernel here")


# speedup vs baseline: 8.2721x; 8.2721x over previous
import functools

import jax
import jax.numpy as jnp
from jax import lax
from jax.experimental import pallas as pl
from jax.experimental.pallas import tpu as pltpu
from jax.experimental.pallas import tpu_sc as plsc

N_NODES = 10000
N_EDGES = 320000
D = 128

NUM_CORES = 2
NUM_SUBCORES = 16
NW = NUM_CORES * NUM_SUBCORES
EPW = N_EDGES // NW
B = 80
NCHUNK = EPW // B
CPY = 400

_mesh = plsc.VectorSubcoreMesh(core_axis_name="c", subcore_axis_name="s")


@functools.partial(
    pl.kernel,
    out_type=jax.ShapeDtypeStruct((N_NODES + N_EDGES, D), jnp.float32),
    mesh=_mesh,
    scratch_types=[
        pltpu.VMEM((EPW,), jnp.int32),
        pltpu.VMEM((EPW,), jnp.int32),
        pltpu.VMEM((B, D), jnp.float32),
        pltpu.VMEM((B, D), jnp.float32),
        pltpu.VMEM((B, D), jnp.float32),
        pltpu.SemaphoreType.DMA,
        pltpu.SemaphoreType.DMA,
    ],
)
def _pool_kernel(x, i0, i1, out, idx0_v, idx1_v, buf_a, buf_b, buf_o,
                 sem_a, sem_b):
    w = lax.axis_index("s") * NUM_CORES + lax.axis_index("c")

    @pl.when(w < N_NODES // CPY)
    def _copy_input():
        base = w * CPY
        pltpu.sync_copy(x.at[pl.ds(base, CPY)], out.at[pl.ds(base, CPY)])

    ebase = w * EPW
    pltpu.sync_copy(i0.at[pl.ds(ebase, EPW)], idx0_v)
    pltpu.sync_copy(i1.at[pl.ds(ebase, EPW)], idx1_v)

    def chunk(c, carry):
        off = c * B
        ca = pltpu.async_copy(x.at[idx0_v.at[pl.ds(off, B)]], buf_a, sem_a)
        cb = pltpu.async_copy(x.at[idx1_v.at[pl.ds(off, B)]], buf_b, sem_b)
        ca.wait()
        cb.wait()

        def row(r, rc):
            for j in range(8):
                sl = pl.ds(j * 16, 16)
                buf_o[r, sl] = (buf_a[r, sl] + buf_b[r, sl]) * 0.5
            return rc

        lax.fori_loop(0, B, row, None)
        pltpu.sync_copy(buf_o, out.at[pl.ds(N_NODES + ebase + off, B)])
        return carry

    lax.fori_loop(0, NCHUNK, chunk, None)


def kernel(input, pool_idx):
    i0 = pool_idx[:, 0].astype(jnp.int32)
    i1 = pool_idx[:, 1].astype(jnp.int32)
    return _pool_kernel(input, i0, i1)
